# baseline (device time: 46965 ns/iter reference)
import functools

import jax
import jax.numpy as jnp
from jax import lax
from jax.experimental import pallas as pl
from jax.experimental.pallas import tpu as pltpu

N_DEV = 16
N_GROUPS = 4


def kernel(x, W1, W2):
    m, _ = x.shape
    out_n = W2.shape[1]
    rows = m // N_DEV
    grp = m // N_GROUPS
    per_grp = N_DEV // N_GROUPS

    def body(x_ref, w1_ref, w2_ref, out_ref, partial_ref, acc_ref, acc_bf,
             rs_buf, rs_send, rs_recv, ag_send, ag_recv):
        my_i = lax.axis_index("i")

        xb = x_ref[...].astype(jnp.bfloat16)
        w1 = w1_ref[...].astype(jnp.bfloat16)
        w2 = w2_ref[...].astype(jnp.bfloat16)

        rs = []
        for g in range(N_GROUPS):
            hg = jnp.dot(xb[g * grp:(g + 1) * grp, :], w1,
                         preferred_element_type=jnp.float32)
            hg = jnp.maximum(hg, 0.0).astype(jnp.bfloat16)
            pg = jnp.dot(hg, w2, preferred_element_type=jnp.float32)
            partial_ref[pl.ds(g * grp, grp), :] = pg.astype(jnp.bfloat16)
            for j in range(g * per_grp, (g + 1) * per_grp):
                jl = j - g * per_grp
                @pl.when(j == my_i)
                def _(pg=pg, jl=jl):
                    acc_ref[...] = pg[jl * rows:(jl + 1) * rows, :]

                r = pltpu.make_async_remote_copy(
                    src_ref=partial_ref.at[pl.ds(j * rows, rows)],
                    dst_ref=rs_buf.at[my_i],
                    send_sem=rs_send.at[j],
                    recv_sem=rs_recv.at[my_i],
                    device_id=(j,),
                    device_id_type=pl.DeviceIdType.MESH,
                )

                @pl.when(j != my_i)
                def _(r=r):
                    r.start()

                rs.append(r)

        for j in range(N_DEV):
            rcv = pltpu.make_async_remote_copy(
                src_ref=partial_ref.at[pl.ds(0, rows)],
                dst_ref=rs_buf.at[j],
                send_sem=rs_send.at[j],
                recv_sem=rs_recv.at[j],
                device_id=(j,),
                device_id_type=pl.DeviceIdType.MESH,
            )

            @pl.when(j != my_i)
            def _(rcv=rcv, j=j):
                rcv.wait_recv()
                acc_ref[...] += rs_buf[j].astype(jnp.float32)

        acc_bf[...] = acc_ref[...].astype(jnp.bfloat16)

        ag = []
        for j in range(N_DEV):
            r = pltpu.make_async_remote_copy(
                src_ref=acc_bf,
                dst_ref=out_ref.at[pl.ds(my_i * rows, rows)],
                send_sem=ag_send.at[j],
                recv_sem=ag_recv.at[my_i],
                device_id=(j,),
                device_id_type=pl.DeviceIdType.MESH,
            )

            @pl.when(j != my_i)
            def _(r=r):
                r.start()

            ag.append(r)

        out_ref[pl.ds(my_i * rows, rows), :] = acc_bf[...]

        for j in range(N_DEV):
            rcv = pltpu.make_async_remote_copy(
                src_ref=acc_bf,
                dst_ref=out_ref.at[pl.ds(j * rows, rows)],
                send_sem=ag_send.at[j],
                recv_sem=ag_recv.at[j],
                device_id=(j,),
                device_id_type=pl.DeviceIdType.MESH,
            )

            @pl.when(j != my_i)
            def _(rcv=rcv, sj=rs[j], aj=ag[j]):
                rcv.wait_recv()
                sj.wait_send()
                aj.wait_send()

    return pl.pallas_call(
        body,
        out_shape=jax.ShapeDtypeStruct((m, out_n), jnp.bfloat16),
        in_specs=[pl.BlockSpec(memory_space=pltpu.VMEM)] * 3,
        out_specs=pl.BlockSpec(memory_space=pltpu.VMEM),
        scratch_shapes=[
            pltpu.VMEM((m, out_n), jnp.bfloat16),
            pltpu.VMEM((rows, out_n), jnp.float32),
            pltpu.VMEM((rows, out_n), jnp.bfloat16),
            pltpu.VMEM((N_DEV, rows, out_n), jnp.bfloat16),
            pltpu.SemaphoreType.DMA((N_DEV,)),
            pltpu.SemaphoreType.DMA((N_DEV,)),
            pltpu.SemaphoreType.DMA((N_DEV,)),
            pltpu.SemaphoreType.DMA((N_DEV,)),
        ],
    )(x, W1, W2)


# device time: 45511 ns/iter; 1.0319x vs baseline; 1.0319x over previous
import jax
import jax.numpy as jnp
from jax import lax
from jax.experimental import pallas as pl
from jax.experimental.pallas import tpu as pltpu

N_DEV = 16


def kernel(x, W1, W2):
    m, _ = x.shape
    out_n = W2.shape[1]
    rows = m // N_DEV

    x = x.astype(jnp.bfloat16)
    W1 = W1.astype(jnp.bfloat16)
    W2 = W2.astype(jnp.bfloat16)

    def body(x_ref, w1_ref, w2_ref, out_ref, partial_ref, acc_ref, rs_buf,
             rs_send, rs_recv, ag_send, ag_recv):
        my_i = lax.axis_index("i")

        h = jnp.dot(x_ref[...], w1_ref[...],
                    preferred_element_type=jnp.float32)
        hb = jnp.maximum(h, 0.0).astype(jnp.bfloat16)
        partial_f32 = jnp.dot(hb, w2_ref[...],
                              preferred_element_type=jnp.float32)
        partial_ref[...] = partial_f32.astype(jnp.bfloat16)

        rs = []
        for d in range(1, N_DEV):
            tgt = (my_i + d) % N_DEV
            r = pltpu.make_async_remote_copy(
                src_ref=partial_ref.at[pl.ds(tgt * rows, rows)],
                dst_ref=rs_buf.at[d - 1],
                send_sem=rs_send.at[d - 1],
                recv_sem=rs_recv.at[d - 1],
                device_id=(tgt,),
                device_id_type=pl.DeviceIdType.MESH,
            )
            r.start()
            rs.append(r)

        acc = partial_ref[pl.ds(my_i * rows, rows), :].astype(jnp.float32)
        for d in range(1, N_DEV):
            rs[d - 1].wait_recv()
            acc = acc + rs_buf[d - 1].astype(jnp.float32)
        acc_ref[...] = acc.astype(jnp.bfloat16)

        ag = []
        for d in range(1, N_DEV):
            tgt = (my_i + d) % N_DEV
            r = pltpu.make_async_remote_copy(
                src_ref=acc_ref,
                dst_ref=out_ref.at[pl.ds(my_i * rows, rows)],
                send_sem=ag_send.at[d - 1],
                recv_sem=ag_recv.at[d - 1],
                device_id=(tgt,),
                device_id_type=pl.DeviceIdType.MESH,
            )
            r.start()
            ag.append(r)

        out_ref[pl.ds(my_i * rows, rows), :] = acc_ref[...]

        for d in range(1, N_DEV):
            ag[d - 1].wait_recv()
            rs[d - 1].wait_send()
            ag[d - 1].wait_send()

    return pl.pallas_call(
        body,
        out_shape=jax.ShapeDtypeStruct((m, out_n), jnp.bfloat16),
        in_specs=[pl.BlockSpec(memory_space=pltpu.VMEM)] * 3,
        out_specs=pl.BlockSpec(memory_space=pltpu.VMEM),
        scratch_shapes=[
            pltpu.VMEM((m, out_n), jnp.bfloat16),
            pltpu.VMEM((rows, out_n), jnp.bfloat16),
            pltpu.VMEM((N_DEV - 1, rows, out_n), jnp.bfloat16),
            pltpu.SemaphoreType.DMA((N_DEV - 1,)),
            pltpu.SemaphoreType.DMA((N_DEV - 1,)),
            pltpu.SemaphoreType.DMA((N_DEV - 1,)),
            pltpu.SemaphoreType.DMA((N_DEV - 1,)),
        ],
    )(x, W1, W2)


# device time: 32621 ns/iter; 1.4397x vs baseline; 1.3951x over previous
import jax
import jax.numpy as jnp
from jax import lax
from jax.experimental import pallas as pl
from jax.experimental.pallas import tpu as pltpu

N_DEV = 16
NS = 2


def kernel(x, W1, W2):
    m, _ = x.shape
    out_n = W2.shape[1]
    rows = m // N_DEV
    cols = out_n // NS

    def body(x_ref, w1_ref, w2_ref, out_ref, partial_ref, acc_ref,
             rs_buf, rs_send, rs_recv, ag_send, ag_recv):
        my_i = lax.axis_index("i")

        xb = x_ref[...].astype(jnp.bfloat16)
        w1 = w1_ref[...].astype(jnp.bfloat16)
        h = jnp.dot(xb, w1, preferred_element_type=jnp.float32)
        hb = jnp.maximum(h, 0.0).astype(jnp.bfloat16)

        rs = [[None] * (N_DEV - 1) for _ in range(NS)]
        for s in range(NS):
            w2s = w2_ref[:, s * cols:(s + 1) * cols].astype(jnp.bfloat16)
            ps = jnp.dot(hb, w2s, preferred_element_type=jnp.float32)
            partial_ref[:, pl.ds(s * cols, cols)] = ps.astype(jnp.bfloat16)
            for d in range(1, N_DEV):
                tgt = (my_i + d) % N_DEV
                r = pltpu.make_async_remote_copy(
                    src_ref=partial_ref.at[pl.ds(tgt * rows, rows),
                                           pl.ds(s * cols, cols)],
                    dst_ref=rs_buf.at[s, d - 1],
                    send_sem=rs_send.at[s, d - 1],
                    recv_sem=rs_recv.at[s, d - 1],
                    device_id=(tgt,),
                    device_id_type=pl.DeviceIdType.MESH,
                )
                r.start()
                rs[s][d - 1] = r

        ag = [[None] * (N_DEV - 1) for _ in range(NS)]
        for s in range(NS):
            acc = partial_ref[pl.ds(my_i * rows, rows),
                              pl.ds(s * cols, cols)].astype(jnp.float32)
            for d in range(1, N_DEV):
                rs[s][d - 1].wait_recv()
                acc = acc + rs_buf[s, d - 1].astype(jnp.float32)
            acc_ref[:, pl.ds(s * cols, cols)] = acc.astype(jnp.bfloat16)
            for d in range(1, N_DEV):
                tgt = (my_i + d) % N_DEV
                r = pltpu.make_async_remote_copy(
                    src_ref=acc_ref.at[:, pl.ds(s * cols, cols)],
                    dst_ref=out_ref.at[pl.ds(my_i * rows, rows),
                                       pl.ds(s * cols, cols)],
                    send_sem=ag_send.at[s, d - 1],
                    recv_sem=ag_recv.at[s, d - 1],
                    device_id=(tgt,),
                    device_id_type=pl.DeviceIdType.MESH,
                )
                r.start()
                ag[s][d - 1] = r
            out_ref[pl.ds(my_i * rows, rows), pl.ds(s * cols, cols)] = (
                acc_ref[:, pl.ds(s * cols, cols)])

        for s in range(NS):
            for d in range(1, N_DEV):
                ag[s][d - 1].wait_recv()
                rs[s][d - 1].wait_send()
                ag[s][d - 1].wait_send()

    return pl.pallas_call(
        body,
        out_shape=jax.ShapeDtypeStruct((m, out_n), jnp.bfloat16),
        in_specs=[pl.BlockSpec(memory_space=pltpu.VMEM)] * 3,
        out_specs=pl.BlockSpec(memory_space=pltpu.VMEM),
        scratch_shapes=[
            pltpu.VMEM((m, out_n), jnp.bfloat16),
            pltpu.VMEM((rows, out_n), jnp.bfloat16),
            pltpu.VMEM((NS, N_DEV - 1, rows, cols), jnp.bfloat16),
            pltpu.SemaphoreType.DMA((NS, N_DEV - 1)),
            pltpu.SemaphoreType.DMA((NS, N_DEV - 1)),
            pltpu.SemaphoreType.DMA((NS, N_DEV - 1)),
            pltpu.SemaphoreType.DMA((NS, N_DEV - 1)),
        ],
    )(x, W1, W2)
